# baseline (speedup 1.0000x reference)
import functools

import jax
import jax.numpy as jnp
from jax import lax
from jax.experimental import pallas as pl
from jax.experimental.pallas import tpu as pltpu
from jax.experimental.pallas import tpu_sc as plsc

_B = 512
_S = 1024
_R1 = 8192
_R2 = 24576
_NS = 256
_NM = 256
_NSMT = 4096
_LF = 1e-6
_NAL = 2.0
_EPS = 1e-30

_C1 = 1024
_C2 = 1024
_HI = (_R1 + _R2) // 128

_NG = _R1 + 2 * _R2
_NC = 2 * _R1 + 3 * _R2
_NW = 32
_RPW = _NG // _NW
_RCH = 64
_NCH = _RPW // _RCH
_EPW = _NC // _NW
_ECH = 128

_INTERPRET = False


def _sigmoid(x):
    return 1.0 / (1.0 + jnp.exp(-x))


def _med(t):
    Tg = 10.0 + 290.0 * _sigmoid(1e-3 * t)
    return jnp.log(Tg / 300.0), 1.0 / Tg


def _make_sc_gather():
    mesh = plsc.VectorSubcoreMesh(core_axis_name="c", subcore_axis_name="s")

    @functools.partial(
        pl.kernel,
        mesh=mesh,
        out_type=[
            jax.ShapeDtypeStruct((_NG, _B), jnp.float32),
            jax.ShapeDtypeStruct((_NC,), jnp.float32),
        ],
        scratch_types=[
            pltpu.VMEM((_RCH,), jnp.int32),
            pltpu.VMEM((_RCH, _B), jnp.float32),
            pltpu.VMEM((_ECH,), jnp.int32),
            pltpu.VMEM((_ECH,), jnp.float32),
            pltpu.SemaphoreType.DMA,
        ],
    )
    def _sc_gather_kernel(yt_hbm, gidx_hbm, cnt_hbm, cidx_hbm, yg_hbm, cg_hbm,
                          idx_v, rows_v, cidx_v, cg_v, sem):
        wid = lax.axis_index("s") * 2 + lax.axis_index("c")
        base = wid * _RPW

        def body(i, carry):
            off = base + i * _RCH
            pltpu.sync_copy(gidx_hbm.at[pl.ds(off, _RCH)], idx_v)
            pltpu.async_copy(yt_hbm.at[idx_v], rows_v, sem).wait()
            pltpu.sync_copy(rows_v, yg_hbm.at[pl.ds(off, _RCH)])
            return carry

        lax.fori_loop(0, _NCH, body, 0)

        cbase = wid * _EPW

        def cbody(j, carry):
            off = cbase + j * _ECH
            pltpu.sync_copy(cidx_hbm.at[pl.ds(off, _ECH)], cidx_v)
            pltpu.async_copy(cnt_hbm.at[cidx_v], cg_v, sem).wait()
            pltpu.sync_copy(cg_v, cg_hbm.at[pl.ds(off, _ECH)])
            return carry

        lax.fori_loop(0, _EPW // _ECH, cbody, 0)

    return _sc_gather_kernel


def _gather_all(y_t, gidx, cnt1d, cidx):
    return _make_sc_gather()(y_t, gidx, cnt1d, cidx)


def _setup_kernel(yt_ref, surf_ref, mant_ref, smtr_ref, smtc_ref,
                  cnt_ref, ys_ref, ym_ref, kmat_ref):
    iota_s = lax.broadcasted_iota(jnp.int32, (_S, _NS), 0)
    cnt = jnp.sum((iota_s == surf_ref[...]).astype(jnp.float32),
                  axis=1, keepdims=True)
    cntm = jnp.sum((iota_s == mant_ref[...]).astype(jnp.float32),
                   axis=1, keepdims=True)
    cnt_ref[...] = cnt
    yt = yt_ref[...]
    ys_ref[...] = jnp.sum(yt * cnt, axis=0, keepdims=True)
    ym_ref[...] = jnp.sum(yt * cntm, axis=0, keepdims=True)
    hi_row = smtr_ref[...] // 128
    lo_col = smtc_ref[...] % 128
    mh = (lax.broadcasted_iota(jnp.int32, (_HI, _NSMT), 0)
          == hi_row).astype(jnp.float32)
    ml = (lax.broadcasted_iota(jnp.int32, (_NSMT, 128), 1)
          == lo_col).astype(jnp.float32)
    kmat_ref[...] = jnp.dot(mh, ml, preferred_element_type=jnp.float32)


def _p1t_kernel(t_ref, ya_ref, a_ref, b_ref, g_ref, cp_ref, cr_ref,
                ra_ref, net_ref):
    i = pl.program_id(0)

    @pl.when(i == 0)
    def _init():
        net_ref[...] = jnp.zeros_like(net_ref)

    L, invT = _med(t_ref[...])
    c = a_ref[...] * jnp.exp(b_ref[...] * L - g_ref[...] * invT)
    ra = c * ya_ref[...]
    ra_ref[...] = ra
    w = cp_ref[...] - cr_ref[...]
    net_ref[...] += jnp.sum(ra * w, axis=0, keepdims=True)


def _p2t_kernel(t_ref, yb1_ref, yb2_ref, a_ref, b_ref, g_ref,
                cp_ref, cra_ref, crb_ref, rb_ref, net_ref):
    i = pl.program_id(0)

    @pl.when(i == 0)
    def _init():
        net_ref[...] = jnp.zeros_like(net_ref)

    t = t_ref[...]
    L, invT = _med(t)
    den = jnp.exp(4.0 + 2.0 * jnp.tanh(5e-4 * t))
    c = a_ref[...] * jnp.exp(b_ref[...] * L - g_ref[...] * invT)
    rb = c * yb1_ref[...] * yb2_ref[...] * den
    rb_ref[...] = rb
    w = cp_ref[...] - cra_ref[...] - crb_ref[...]
    net_ref[...] += jnp.sum(rb * w, axis=0, keepdims=True)


def _scale(net1, net2, ys, ym):
    nl = _LF * (ys + ym)
    decay = jnp.minimum(_NAL / (nl + _EPS), 1.0)
    sc = decay * _sigmoid(net1 + net2)
    return jnp.log(sc)


def _s1t_kernel(ra_ref, p1_ref, r11_ref, k_ref, n1_ref, n2_ref,
                ys_ref, ym_ref, dy_ref):
    i = pl.program_id(0)

    @pl.when(i == 0)
    def _init():
        dy_ref[...] = jnp.zeros_like(dy_ref)

    lsc = _scale(n1_ref[...], n2_ref[...], ys_ref[...], ym_ref[...])
    rs = ra_ref[...] * jnp.exp(k_ref[...] * lsc)
    iota_sub = lax.broadcasted_iota(jnp.int32, (_S, _C1), 0)
    M = ((iota_sub == p1_ref[...]).astype(jnp.float32)
         - (iota_sub == r11_ref[...]).astype(jnp.float32))
    dy_ref[...] += jnp.dot(M, rs, preferred_element_type=jnp.float32)


def _s2t_kernel(rb_ref, p2_ref, r12_ref, r22_ref, k_ref, n1_ref, n2_ref,
                ys_ref, ym_ref, dy1_ref, dy_ref):
    i = pl.program_id(0)

    @pl.when(i == 0)
    def _init():
        dy_ref[...] = dy1_ref[...]

    lsc = _scale(n1_ref[...], n2_ref[...], ys_ref[...], ym_ref[...])
    rs = rb_ref[...] * jnp.exp(k_ref[...] * lsc)
    iota_sub = lax.broadcasted_iota(jnp.int32, (_S, _C2), 0)
    M = ((iota_sub == p2_ref[...]).astype(jnp.float32)
         - (iota_sub == r12_ref[...]).astype(jnp.float32)
         - (iota_sub == r22_ref[...]).astype(jnp.float32))
    dy_ref[...] += jnp.dot(M, rs, preferred_element_type=jnp.float32)


def _row(x, n):
    return x.astype(jnp.int32).reshape(1, n)


def _col(x, n):
    return x.astype(jnp.int32).reshape(n, 1)


def kernel(t_in, y_in, alpha_1st, beta_1st, gamma_1st, alpha_2nd, beta_2nd,
           gamma_2nd, r1_1st, p_1st, r1_2nd, r2_2nd, p_2nd,
           inds_surf, inds_mant, inds_smt):
    f32 = jnp.float32
    i32 = jnp.int32
    t_row = t_in.astype(f32).reshape(1, _B)
    y_t = y_in.astype(f32).T
    a1 = alpha_1st.astype(f32).reshape(_R1, 1)
    b1 = beta_1st.astype(f32).reshape(_R1, 1)
    g1 = gamma_1st.astype(f32).reshape(_R1, 1)
    a2 = alpha_2nd.astype(f32).reshape(_R2, 1)
    b2 = beta_2nd.astype(f32).reshape(_R2, 1)
    g2 = gamma_2nd.astype(f32).reshape(_R2, 1)

    const = lambda *bs: pl.BlockSpec(bs, lambda i: (0,) * len(bs))
    rowblk = lambda c: pl.BlockSpec((1, c), lambda i: (0, i))
    colblk = lambda c: pl.BlockSpec((c, 1), lambda i: (i, 0))

    cnt, ysurf, ymant, kmat = pl.pallas_call(
        _setup_kernel,
        grid=(1,),
        in_specs=[
            const(_S, _B), const(1, _NS), const(1, _NM),
            const(1, _NSMT), const(_NSMT, 1),
        ],
        out_specs=[
            const(_S, 1), const(1, _B), const(1, _B), const(_HI, 128),
        ],
        out_shape=[
            jax.ShapeDtypeStruct((_S, 1), f32),
            jax.ShapeDtypeStruct((1, _B), f32),
            jax.ShapeDtypeStruct((1, _B), f32),
            jax.ShapeDtypeStruct((_HI, 128), f32),
        ],
        interpret=_INTERPRET,
    )(y_t, _row(inds_surf, _NS), _row(inds_mant, _NM),
      _row(inds_smt, _NSMT), _col(inds_smt, _NSMT))

    gidx = jnp.concatenate([r1_1st, r1_2nd, r2_2nd]).astype(i32)
    cidx = jnp.concatenate([r1_1st, p_1st, r1_2nd, r2_2nd,
                            p_2nd]).astype(i32)
    cnt1d = cnt.reshape(_S)

    yg, cg = _gather_all(y_t, gidx, cnt1d, cidx)

    yA = yg[:_R1]
    yB1 = yg[_R1:_R1 + _R2]
    yB2 = yg[_R1 + _R2:]
    cr11 = cg[:_R1].reshape(_R1, 1)
    cp1 = cg[_R1:2 * _R1].reshape(_R1, 1)
    cr12 = cg[2 * _R1:2 * _R1 + _R2].reshape(_R2, 1)
    cr22 = cg[2 * _R1 + _R2:2 * _R1 + 2 * _R2].reshape(_R2, 1)
    cp2 = cg[2 * _R1 + 2 * _R2:].reshape(_R2, 1)

    n1 = _R1 // _C1
    ra, net1 = pl.pallas_call(
        _p1t_kernel,
        grid=(n1,),
        in_specs=[
            const(1, _B), pl.BlockSpec((_C1, _B), lambda i: (i, 0)),
            colblk(_C1), colblk(_C1), colblk(_C1),
            colblk(_C1), colblk(_C1),
        ],
        out_specs=[
            pl.BlockSpec((_C1, _B), lambda i: (i, 0)),
            const(1, _B),
        ],
        out_shape=[
            jax.ShapeDtypeStruct((_R1, _B), f32),
            jax.ShapeDtypeStruct((1, _B), f32),
        ],
        interpret=_INTERPRET,
    )(t_row, yA, a1, b1, g1, cp1, cr11)

    n2 = _R2 // _C2
    rb, net2 = pl.pallas_call(
        _p2t_kernel,
        grid=(n2,),
        in_specs=[
            const(1, _B),
            pl.BlockSpec((_C2, _B), lambda i: (i, 0)),
            pl.BlockSpec((_C2, _B), lambda i: (i, 0)),
            colblk(_C2), colblk(_C2), colblk(_C2),
            colblk(_C2), colblk(_C2), colblk(_C2),
        ],
        out_specs=[
            pl.BlockSpec((_C2, _B), lambda i: (i, 0)),
            const(1, _B),
        ],
        out_shape=[
            jax.ShapeDtypeStruct((_R2, _B), f32),
            jax.ShapeDtypeStruct((1, _B), f32),
        ],
        interpret=_INTERPRET,
    )(t_row, yB1, yB2, a2, b2, g2, cp2, cr12, cr22)

    k_col = kmat.reshape(_R1 + _R2, 1)
    k1 = k_col[:_R1]
    k2 = k_col[_R1:]

    dy1 = pl.pallas_call(
        _s1t_kernel,
        grid=(n1,),
        in_specs=[
            pl.BlockSpec((_C1, _B), lambda i: (i, 0)),
            rowblk(_C1), rowblk(_C1), colblk(_C1),
            const(1, _B), const(1, _B), const(1, _B), const(1, _B),
        ],
        out_specs=const(_S, _B),
        out_shape=jax.ShapeDtypeStruct((_S, _B), f32),
        interpret=_INTERPRET,
    )(ra, _row(p_1st, _R1), _row(r1_1st, _R1), k1, net1, net2, ysurf, ymant)

    dy_t = pl.pallas_call(
        _s2t_kernel,
        grid=(n2,),
        in_specs=[
            pl.BlockSpec((_C2, _B), lambda i: (i, 0)),
            rowblk(_C2), rowblk(_C2), rowblk(_C2), colblk(_C2),
            const(1, _B), const(1, _B), const(1, _B), const(1, _B),
            const(_S, _B),
        ],
        out_specs=const(_S, _B),
        out_shape=jax.ShapeDtypeStruct((_S, _B), f32),
        interpret=_INTERPRET,
    )(rb, _row(p_2nd, _R2), _row(r1_2nd, _R2), _row(r2_2nd, _R2), k2,
      net1, net2, ysurf, ymant, dy1)

    return dy_t.T


# speedup vs baseline: 2.3391x; 2.3391x over previous
import jax
import jax.numpy as jnp
from jax.experimental import pallas as pl

_B = 512
_S = 1024
_R1 = 8192
_R2 = 24576
_NS = 256
_NM = 256
_NSMT = 4096
_LF = 1e-6
_NAL = 2.0
_EPS = 1e-30

_C1 = 1024
_C2 = 1024
_HI = (_R1 + _R2) // 128

_INTERPRET = False


def _sigmoid(x):
    return 1.0 / (1.0 + jnp.exp(-x))


def _med(t_col):
    Tg = 10.0 + 290.0 * _sigmoid(1e-3 * t_col)
    return jnp.log(Tg / 300.0), 1.0 / Tg


def _p1st_kernel(t_ref, y_ref, a_ref, b_ref, g_ref, r11_ref, p1_ref,
                 surf_ref, mant_ref, smtr_ref, smtc_ref,
                 ra_ref, net_ref, ys_ref, ym_ref, cnt_ref, kmat_ref):
    i = pl.program_id(0)

    @pl.when(i == 0)
    def _init():
        iota_s = jax.lax.broadcasted_iota(jnp.int32, (_S, _NS), 0)
        cnt = jnp.sum((iota_s == surf_ref[...]).astype(jnp.float32),
                      axis=1, keepdims=True)
        cntm = jnp.sum((iota_s == mant_ref[...]).astype(jnp.float32),
                       axis=1, keepdims=True)
        cnt_ref[...] = cnt
        ys_ref[...] = jnp.dot(y_ref[...], cnt,
                              preferred_element_type=jnp.float32)
        ym_ref[...] = jnp.dot(y_ref[...], cntm,
                              preferred_element_type=jnp.float32)
        hi_row = smtr_ref[...] // 128
        lo_col = smtc_ref[...] % 128
        mh = (jax.lax.broadcasted_iota(jnp.int32, (_HI, _NSMT), 0)
              == hi_row).astype(jnp.float32)
        ml = (jax.lax.broadcasted_iota(jnp.int32, (_NSMT, 128), 1)
              == lo_col).astype(jnp.float32)
        kmat_ref[...] = jnp.dot(mh, ml, preferred_element_type=jnp.float32)
        net_ref[...] = jnp.zeros_like(net_ref)

    L, invT = _med(t_ref[...])
    c = a_ref[...] * jnp.exp(b_ref[...] * L - g_ref[...] * invT)
    iota_sub = jax.lax.broadcasted_iota(jnp.int32, (_S, _C1), 0)
    G = (iota_sub == r11_ref[...]).astype(jnp.float32)
    P = (iota_sub == p1_ref[...]).astype(jnp.float32)
    yA = jnp.dot(y_ref[...], G, preferred_element_type=jnp.float32)
    ra = c * yA
    ra_ref[...] = ra
    w = jnp.sum((P - G) * cnt_ref[...], axis=0, keepdims=True)
    net_ref[...] += jnp.sum(ra * w, axis=1, keepdims=True)


def _p2nd_kernel(t_ref, y_ref, a_ref, b_ref, g_ref, r12_ref, r22_ref, p2_ref,
                 cnt_ref, rb_ref, net_ref):
    i = pl.program_id(0)

    @pl.when(i == 0)
    def _init():
        net_ref[...] = jnp.zeros_like(net_ref)

    t = t_ref[...]
    L, invT = _med(t)
    den = jnp.exp(4.0 + 2.0 * jnp.tanh(5e-4 * t))
    c = a_ref[...] * jnp.exp(b_ref[...] * L - g_ref[...] * invT)
    iota_sub = jax.lax.broadcasted_iota(jnp.int32, (_S, _C2), 0)
    Ga = (iota_sub == r12_ref[...]).astype(jnp.float32)
    Gb = (iota_sub == r22_ref[...]).astype(jnp.float32)
    P = (iota_sub == p2_ref[...]).astype(jnp.float32)
    yB1 = jnp.dot(y_ref[...], Ga, preferred_element_type=jnp.float32)
    yB2 = jnp.dot(y_ref[...], Gb, preferred_element_type=jnp.float32)
    rb = c * yB1 * yB2 * den
    rb_ref[...] = rb
    w = jnp.sum((P - Ga - Gb) * cnt_ref[...], axis=0, keepdims=True)
    net_ref[...] += jnp.sum(rb * w, axis=1, keepdims=True)


def _scale(net1, net2, ys, ym):
    nl = _LF * (ys + ym)
    decay = jnp.minimum(_NAL / (nl + _EPS), 1.0)
    sc = decay * _sigmoid(net1 + net2)
    return jnp.log(sc)


def _s1st_kernel(ra_ref, p1_ref, r11_ref, k_ref, n1_ref, n2_ref,
                 ys_ref, ym_ref, dy_ref):
    i = pl.program_id(0)

    @pl.when(i == 0)
    def _init():
        dy_ref[...] = jnp.zeros_like(dy_ref)

    lsc = _scale(n1_ref[...], n2_ref[...], ys_ref[...], ym_ref[...])
    S1 = jnp.exp(k_ref[...] * lsc)
    rs = ra_ref[...] * S1
    iota_lane = jax.lax.broadcasted_iota(jnp.int32, (_C1, _S), 1)
    M = ((iota_lane == p1_ref[...]).astype(jnp.float32)
         - (iota_lane == r11_ref[...]).astype(jnp.float32))
    dy_ref[...] += jnp.dot(rs, M, preferred_element_type=jnp.float32)


def _s2nd_kernel(rb_ref, p2_ref, r12_ref, r22_ref, k_ref, n1_ref, n2_ref,
                 ys_ref, ym_ref, dy1_ref, dy_ref):
    i = pl.program_id(0)

    @pl.when(i == 0)
    def _init():
        dy_ref[...] = dy1_ref[...]

    lsc = _scale(n1_ref[...], n2_ref[...], ys_ref[...], ym_ref[...])
    S2 = jnp.exp(k_ref[...] * lsc)
    rs = rb_ref[...] * S2
    iota_lane = jax.lax.broadcasted_iota(jnp.int32, (_C2, _S), 1)
    M = ((iota_lane == p2_ref[...]).astype(jnp.float32)
         - (iota_lane == r12_ref[...]).astype(jnp.float32)
         - (iota_lane == r22_ref[...]).astype(jnp.float32))
    dy_ref[...] += jnp.dot(rs, M, preferred_element_type=jnp.float32)


def _row(x, n):
    return x.astype(jnp.int32).reshape(1, n)


def _col(x, n):
    return x.astype(jnp.int32).reshape(n, 1)


def kernel(t_in, y_in, alpha_1st, beta_1st, gamma_1st, alpha_2nd, beta_2nd,
           gamma_2nd, r1_1st, p_1st, r1_2nd, r2_2nd, p_2nd,
           inds_surf, inds_mant, inds_smt):
    f32 = jnp.float32
    t_col = t_in.astype(f32).reshape(_B, 1)
    y = y_in.astype(f32)
    a1 = alpha_1st.astype(f32).reshape(1, _R1)
    b1 = beta_1st.astype(f32).reshape(1, _R1)
    g1 = gamma_1st.astype(f32).reshape(1, _R1)
    a2 = alpha_2nd.astype(f32).reshape(1, _R2)
    b2 = beta_2nd.astype(f32).reshape(1, _R2)
    g2 = gamma_2nd.astype(f32).reshape(1, _R2)

    const = lambda *bs: pl.BlockSpec(bs, lambda i: (0,) * len(bs))
    rowblk = lambda c: pl.BlockSpec((1, c), lambda i: (0, i))
    colblk = lambda c: pl.BlockSpec((c, 1), lambda i: (i, 0))

    n1 = _R1 // _C1
    ra, net1, ysurf, ymant, cnt, kmat = pl.pallas_call(
        _p1st_kernel,
        grid=(n1,),
        in_specs=[
            const(_B, 1), const(_B, _S),
            rowblk(_C1), rowblk(_C1), rowblk(_C1),
            rowblk(_C1), rowblk(_C1),
            const(1, _NS), const(1, _NM),
            const(1, _NSMT), const(_NSMT, 1),
        ],
        out_specs=[
            pl.BlockSpec((_B, _C1), lambda i: (0, i)),
            const(_B, 1), const(_B, 1), const(_B, 1),
            const(_S, 1), const(_HI, 128),
        ],
        out_shape=[
            jax.ShapeDtypeStruct((_B, _R1), f32),
            jax.ShapeDtypeStruct((_B, 1), f32),
            jax.ShapeDtypeStruct((_B, 1), f32),
            jax.ShapeDtypeStruct((_B, 1), f32),
            jax.ShapeDtypeStruct((_S, 1), f32),
            jax.ShapeDtypeStruct((_HI, 128), f32),
        ],
        interpret=_INTERPRET,
    )(t_col, y, a1, b1, g1, _row(r1_1st, _R1), _row(p_1st, _R1),
      _row(inds_surf, _NS), _row(inds_mant, _NM),
      _row(inds_smt, _NSMT), _col(inds_smt, _NSMT))

    n2 = _R2 // _C2
    rb, net2 = pl.pallas_call(
        _p2nd_kernel,
        grid=(n2,),
        in_specs=[
            const(_B, 1), const(_B, _S),
            rowblk(_C2), rowblk(_C2), rowblk(_C2),
            rowblk(_C2), rowblk(_C2), rowblk(_C2),
            const(_S, 1),
        ],
        out_specs=[
            pl.BlockSpec((_B, _C2), lambda i: (0, i)),
            const(_B, 1),
        ],
        out_shape=[
            jax.ShapeDtypeStruct((_B, _R2), f32),
            jax.ShapeDtypeStruct((_B, 1), f32),
        ],
        interpret=_INTERPRET,
    )(t_col, y, a2, b2, g2, _row(r1_2nd, _R2), _row(r2_2nd, _R2),
      _row(p_2nd, _R2), cnt)

    k_row = kmat.reshape(1, _R1 + _R2)
    k1 = k_row[:, :_R1]
    k2 = k_row[:, _R1:]

    dy1 = pl.pallas_call(
        _s1st_kernel,
        grid=(n1,),
        in_specs=[
            pl.BlockSpec((_B, _C1), lambda i: (0, i)),
            colblk(_C1), colblk(_C1), rowblk(_C1),
            const(_B, 1), const(_B, 1), const(_B, 1), const(_B, 1),
        ],
        out_specs=const(_B, _S),
        out_shape=jax.ShapeDtypeStruct((_B, _S), f32),
        interpret=_INTERPRET,
    )(ra, _col(p_1st, _R1), _col(r1_1st, _R1), k1, net1, net2, ysurf, ymant)

    dy = pl.pallas_call(
        _s2nd_kernel,
        grid=(n2,),
        in_specs=[
            pl.BlockSpec((_B, _C2), lambda i: (0, i)),
            colblk(_C2), colblk(_C2), colblk(_C2), rowblk(_C2),
            const(_B, 1), const(_B, 1), const(_B, 1), const(_B, 1),
            const(_B, _S),
        ],
        out_specs=const(_B, _S),
        out_shape=jax.ShapeDtypeStruct((_B, _S), f32),
        interpret=_INTERPRET,
    )(rb, _col(p_2nd, _R2), _col(r1_2nd, _R2), _col(r2_2nd, _R2), k2,
      net1, net2, ysurf, ymant, dy1)

    return dy
